# baseline (speedup 1.0000x reference)
import functools

import jax
import jax.numpy as jnp
from jax import lax
from jax.experimental import pallas as pl
from jax.experimental.pallas import tpu as pltpu
from jax.experimental.pallas import tpu_sc as plsc

_NC = 2
_NS = 16
_NW = _NC * _NS
_CH = 128
_DEGW = 128
_PH = 2
_SPLIT_FRAC0 = 0.3

_HI = lax.Precision.HIGHEST


def _rup(a, b):
    return (a + b - 1) // b * b


def _sc_hist_body(np_, cw, dst_hbm, out_hbm, idx_v, buf, acc):
    c = lax.axis_index("c")
    s = lax.axis_index("s")
    w = c * _NS + s
    rows = np_ // _NS
    base = s * rows

    def fill0(i, carry):
        for t in range(_DEGW // 16):
            buf[i, pl.ds(t * 16, 16)] = jnp.zeros((16,), jnp.float32)
        return carry

    lax.fori_loop(0, _CH, fill0, 0)
    for k in range(rows // _CH):
        pltpu.sync_copy(buf, acc.at[pl.ds(base + k * _CH, _CH)])
    pltpu.sync_copy(dst_hbm.at[w], idx_v)

    def fill1(i, carry):
        for t in range(_DEGW // 16):
            buf[i, pl.ds(t * 16, 16)] = jnp.ones((16,), jnp.float32)
        return carry

    lax.fori_loop(0, _CH, fill1, 0)
    plsc.subcore_barrier()

    def scat(j, carry):
        pltpu.sync_copy(buf, acc.at[idx_v.at[j]], add=True)
        return carry

    lax.fori_loop(0, cw, scat, 0)
    plsc.subcore_barrier()
    for k in range(rows // _CH):
        pltpu.sync_copy(acc.at[pl.ds(base + k * _CH, _CH)], buf)
        pltpu.sync_copy(buf, out_hbm.at[c, pl.ds(base + k * _CH, _CH)])


def _sc_scatter_body(np_, cw0, cw1, d, xs_hbm, src_hbm, dst_hbm, out_hbm,
                     src_v, dst_v, g0, g1, acc, s0, s1):
    c = lax.axis_index("c")
    s = lax.axis_index("s")
    rows = np_ // _NS
    base = s * rows

    def zero(i, carry):
        for t in range(d // 16):
            g0[i, pl.ds(t * 16, 16)] = jnp.zeros((16,), jnp.float32)
        return carry

    lax.fori_loop(0, _CH, zero, 0)
    for k in range(rows // _CH):
        pltpu.sync_copy(g0, acc.at[pl.ds(base + k * _CH, _CH)])
    plsc.subcore_barrier()

    def run(w, cw_c):
        cwp = cw_c // _PH
        for ph in range(_PH):
            lo = ph * cwp
            pltpu.sync_copy(src_hbm.at[w, pl.ds(lo, cwp)],
                            src_v.at[pl.ds(0, cwp)])
            pltpu.sync_copy(dst_hbm.at[w, pl.ds(lo, cwp)],
                            dst_v.at[pl.ds(0, cwp)])
            pltpu.async_copy(xs_hbm.at[src_v.at[0]], g0, s0)
            pltpu.async_copy(xs_hbm.at[src_v.at[1]], g1, s1)

            def pair(i, carry):
                j = 2 * i
                pltpu.make_async_copy(xs_hbm.at[src_v.at[j]], g0, s0).wait()
                pltpu.sync_copy(g0, acc.at[dst_v.at[j]], add=True)

                @pl.when(j + 2 < cwp)
                def _():
                    pltpu.async_copy(xs_hbm.at[src_v.at[j + 2]], g0, s0)

                pltpu.make_async_copy(xs_hbm.at[src_v.at[j + 1]], g1, s1).wait()
                pltpu.sync_copy(g1, acc.at[dst_v.at[j + 1]], add=True)

                @pl.when(j + 3 < cwp)
                def _():
                    pltpu.async_copy(xs_hbm.at[src_v.at[j + 3]], g1, s1)

                return carry

            lax.fori_loop(0, cwp // 2, pair, 0)

    @pl.when(c == 0)
    def _():
        run(s, cw0)

    @pl.when(c == 1)
    def _():
        run(_NS + s, cw1)

    plsc.subcore_barrier()
    for k in range(rows // _CH):
        pltpu.sync_copy(acc.at[pl.ds(base + k * _CH, _CH)], g0)
        pltpu.sync_copy(g0, out_hbm.at[c, pl.ds(base + k * _CH, _CH)])


def _tc_score_body(xpad_ref, pcol_ref, pn_ref, s_ref):
    pnorm = pn_ref[0, 0]
    xb = xpad_ref[...].astype(jnp.bfloat16)
    pb = pcol_ref[...].astype(jnp.bfloat16)
    raw = lax.dot_general(xb, pb, (((1,), (0,)), ((), ())),
                          preferred_element_type=jnp.float32)
    s_ref[...] = raw / pnorm


def _tc_dense_body(n, rf, x_ref, sc_ref, wih_ref, whh_ref, bih_ref,
                   bhh_ref, w0_ref, xw_ref, s_ref, xt_ref):
    sc = sc_ref[...]
    row_i = lax.broadcasted_iota(jnp.int32, (rf, _CH), 0)
    col_i = lax.broadcasted_iota(jnp.int32, (rf, _CH), 1)
    flat = row_i * _CH + col_i
    neg = jnp.float32(-jnp.inf)
    s_ref[...] = jnp.where(flat < n, sc, neg)

    def pick(k, carry):
        sv = s_ref[...]
        m = jnp.max(sv)
        idx = jnp.min(jnp.where(sv == m, flat, jnp.int32(2**30)))
        row = x_ref[pl.ds(idx, 1), :]
        xt_ref[pl.ds(k, 1), :] = row * jnp.tanh(m)
        s_ref[...] = jnp.where(flat == idx, neg, sv)
        return carry

    lax.fori_loop(0, 128, pick, 0)

    xt = xt_ref[...]
    dn = (((1,), (1,)), ((), ()))
    gi = lax.dot_general(xt, wih_ref[...], dn, precision=_HI) + bih_ref[...]
    gh = lax.dot_general(w0_ref[...], whh_ref[...], dn, precision=_HI) + bhh_ref[...]
    d = xt.shape[1]
    i_r, i_z, i_n = gi[:, :d], gi[:, d:2 * d], gi[:, 2 * d:]
    h_r, h_z, h_n = gh[:, :d], gh[:, d:2 * d], gh[:, 2 * d:]
    r = jax.nn.sigmoid(i_r + h_r)
    z = jax.nn.sigmoid(i_z + h_z)
    nn = jnp.tanh(i_n + r * h_n)
    w = (1.0 - z) * nn + z * w0_ref[...]
    xw_ref[...] = lax.dot_general(x_ref[...], w, (((1,), (0,)), ((), ())),
                                  precision=_HI)


def _tc_scale_body(xw_ref, h0_ref, h1_ref, xs_ref, dis_ref):
    deg = 1.0 + h0_ref[...] + h1_ref[...]
    dis = 1.0 / jnp.sqrt(deg)
    dis_ref[...] = dis
    xs_ref[...] = xw_ref[...] * dis


def _tc_final_body(nb, s2_ref, xs_ref, dis_ref, batch_ref, doc_ref, wdoc_ref,
                   bdoc_ref, bng_ref, bnb_ref, bnm_ref, bnv_ref, wfuse_ref,
                   bfuse_ref, whead_ref, bhead_ref, head_ref):
    s2 = s2_ref[...]
    h = dis_ref[...] * (s2[0] + s2[1] + xs_ref[...])
    bids = lax.broadcasted_iota(jnp.int32, (nb, h.shape[0]), 0)
    mask = (batch_ref[...] == bids).astype(jnp.float32)
    dn = (((1,), (0,)), ((), ()))
    ssum = lax.dot_general(mask, h, dn, precision=_HI)
    cnt = jnp.sum(mask, axis=1, keepdims=True)
    pooled = ssum / jnp.maximum(cnt, 1.0)
    dnt = (((1,), (1,)), ((), ()))
    demb = jnp.maximum(
        lax.dot_general(doc_ref[...], wdoc_ref[...], dnt, precision=_HI)
        + bdoc_ref[...], 0.0)
    z = jnp.concatenate([pooled, demb], axis=1)
    z = (z - bnm_ref[...]) / jnp.sqrt(bnv_ref[...] + 1e-5) * bng_ref[...] \
        + bnb_ref[...]
    z2 = jnp.maximum(
        lax.dot_general(z, wfuse_ref[...], dnt, precision=_HI)
        + bfuse_ref[...], 0.0)
    head_ref[...] = lax.dot_general(z2, whead_ref[...], dnt, precision=_HI) \
        + bhead_ref[...]


def kernel(x, doc_features, p, W_ih, W_hh, b_ih, b_hh, W0, W_doc, b_doc,
           bn_gamma, bn_beta, W_fuse, b_fuse, W_task, b_task, W_time, b_time,
           edge_index, batch, bn_mean, bn_var):
    n, d = x.shape
    nb = doc_features.shape[0]
    e = edge_index.shape[1]
    f32 = jnp.float32

    np_ = _rup(n + 1, _NS * _CH)
    ep = _rup(e, _NW * _CH * _PH * 8)
    cwh = ep // (_NW * _CH)
    rf = _rup(n, _CH) // _CH
    total_ch = 2 * cwh
    cw0 = int(total_ch * _SPLIT_FRAC0 / 16) * 16
    cw1 = total_ch - cw0
    cwmax = max(cw0, cw1)

    srcf = jnp.concatenate(
        [edge_index[0], jnp.full((ep - e,), 0, edge_index.dtype)])
    pad_dst = (n + jnp.arange(ep - e, dtype=edge_index.dtype) % (np_ - n))
    dstf = jnp.concatenate([edge_index[1], pad_dst])
    dst_h = dstf.reshape(_NW, cwh, _CH)

    def _asym(flat):
        a0 = flat[:_NS * cw0 * _CH].reshape(_NS, cw0, _CH)
        a1 = flat[_NS * cw0 * _CH:].reshape(_NS, cw1, _CH)
        a0 = jnp.pad(a0, ((0, 0), (0, cwmax - cw0), (0, 0)))
        a1 = jnp.pad(a1, ((0, 0), (0, cwmax - cw1), (0, 0)))
        return jnp.concatenate([a0, a1], axis=0)

    src = _asym(srcf)
    dst = _asym(dstf)
    xpad = jnp.pad(x, ((0, rf * _CH - n), (0, 0)))
    bih2 = b_ih.reshape(1, -1)
    bhh2 = b_hh.reshape(1, -1)
    batchp = jnp.concatenate(
        [batch, jnp.full((np_ - n,), nb, batch.dtype)]).reshape(1, np_)

    mesh = plsc.VectorSubcoreMesh(core_axis_name="c", subcore_axis_name="s",
                                  num_cores=_NC, num_subcores=_NS)

    hist = pl.kernel(
        functools.partial(_sc_hist_body, np_, cwh),
        out_type=jax.ShapeDtypeStruct((_NC, np_, _DEGW), f32),
        mesh=mesh,
        scratch_types=[
            pltpu.VMEM((cwh, _CH), jnp.int32),
            pltpu.VMEM((_CH, _DEGW), f32),
            pltpu.VMEM_SHARED((np_, _DEGW), f32),
        ],
        name="sc_deg_hist",
    )
    deg2 = hist(dst_h)

    score = pl.pallas_call(
        _tc_score_body,
        out_shape=jax.ShapeDtypeStruct((rf * _CH, 1), f32),
        name="tc_score",
    )
    pn = (jnp.linalg.norm(p) + 1e-16).reshape(1, 1)
    sc_fold = score(xpad, p.reshape(d, 1), pn).reshape(rf, _CH)

    dense = pl.pallas_call(
        functools.partial(_tc_dense_body, n, rf),
        out_shape=jax.ShapeDtypeStruct((n, d), f32),
        scratch_shapes=[
            pltpu.VMEM((rf, _CH), f32),
            pltpu.VMEM((d, d), f32),
        ],
        name="tc_dense",
    )
    xw = dense(x, sc_fold, W_ih, W_hh, bih2, bhh2, W0)

    scale = pl.pallas_call(
        _tc_scale_body,
        out_shape=(jax.ShapeDtypeStruct((n, d), f32),
                   jax.ShapeDtypeStruct((n, 1), f32)),
        name="tc_scale",
    )
    xs, dis = scale(xw, deg2[0, :n, 0:1], deg2[1, :n, 0:1])

    xs_p = jnp.pad(xs, ((0, np_ - n), (0, 0)))
    dis_p = jnp.pad(dis, ((0, np_ - n), (0, 0)))

    scat = pl.kernel(
        functools.partial(_sc_scatter_body, np_, cw0, cw1, d),
        out_type=jax.ShapeDtypeStruct((_NC, np_, d), f32),
        mesh=mesh,
        scratch_types=[
            pltpu.VMEM((cwmax // _PH, _CH), jnp.int32),
            pltpu.VMEM((cwmax // _PH, _CH), jnp.int32),
            pltpu.VMEM((_CH, d), f32),
            pltpu.VMEM((_CH, d), f32),
            pltpu.VMEM_SHARED((np_, d), f32),
            pltpu.SemaphoreType.DMA,
            pltpu.SemaphoreType.DMA,
        ],
        name="sc_edge_scatter",
    )
    s2 = scat(xs_p, src, dst)

    n_task = W_task.shape[0]
    n_time = W_time.shape[0]
    w_head = jnp.concatenate([W_task, W_time], axis=0)
    b_head = jnp.concatenate([b_task, b_time]).reshape(1, -1)
    final = pl.pallas_call(
        functools.partial(_tc_final_body, nb),
        out_shape=jax.ShapeDtypeStruct((nb, n_task + n_time), f32),
        name="tc_final",
    )
    heads = final(s2, xs_p, dis_p, batchp, doc_features, W_doc,
                  b_doc.reshape(1, -1), bn_gamma.reshape(1, -1),
                  bn_beta.reshape(1, -1), bn_mean.reshape(1, -1),
                  bn_var.reshape(1, -1), W_fuse, b_fuse.reshape(1, -1),
                  w_head, b_head)
    return (heads[:, :n_task], heads[:, n_task:])


# speedup vs baseline: 1.0520x; 1.0520x over previous
import functools

import jax
import jax.numpy as jnp
from jax import lax
from jax.experimental import pallas as pl
from jax.experimental.pallas import tpu as pltpu
from jax.experimental.pallas import tpu_sc as plsc

_NC = 2
_NS = 16
_NW = _NC * _NS
_CH = 128
_DEGW = 128
_PH = 1
_SPLIT_FRAC0 = 0.5

_HI = lax.Precision.HIGHEST


def _rup(a, b):
    return (a + b - 1) // b * b


def _sc_hist_body(np_, cw, dst_hbm, out_hbm, idx_v, buf, acc):
    c = lax.axis_index("c")
    s = lax.axis_index("s")
    w = c * _NS + s
    rows = np_ // _NS
    base = s * rows

    def fill0(i, carry):
        for t in range(_DEGW // 16):
            buf[i, pl.ds(t * 16, 16)] = jnp.zeros((16,), jnp.float32)
        return carry

    lax.fori_loop(0, _CH, fill0, 0)
    for k in range(rows // _CH):
        pltpu.sync_copy(buf, acc.at[pl.ds(base + k * _CH, _CH)])
    pltpu.sync_copy(dst_hbm.at[w], idx_v)

    def fill1(i, carry):
        for t in range(_DEGW // 16):
            buf[i, pl.ds(t * 16, 16)] = jnp.ones((16,), jnp.float32)
        return carry

    lax.fori_loop(0, _CH, fill1, 0)
    plsc.subcore_barrier()

    def scat(j, carry):
        pltpu.sync_copy(buf, acc.at[idx_v.at[j]], add=True)
        return carry

    lax.fori_loop(0, cw, scat, 0)
    plsc.subcore_barrier()
    for k in range(rows // _CH):
        pltpu.sync_copy(acc.at[pl.ds(base + k * _CH, _CH)], buf)
        pltpu.sync_copy(buf, out_hbm.at[c, pl.ds(base + k * _CH, _CH)])


def _sc_scatter_body(np_, cw0, cw1, d, xs_hbm, src_hbm, dst_hbm, out_hbm,
                     src_v, dst_v, g0, acc, s0):
    c = lax.axis_index("c")
    s = lax.axis_index("s")
    rows = np_ // _NS
    base = s * rows

    def zero(i, carry):
        for t in range(d // 16):
            g0[i, pl.ds(t * 16, 16)] = jnp.zeros((16,), jnp.float32)
        return carry

    lax.fori_loop(0, _CH, zero, 0)
    for k in range(rows // _CH):
        pltpu.sync_copy(g0, acc.at[pl.ds(base + k * _CH, _CH)])
    plsc.subcore_barrier()

    def run(w, cw_c):
        cwp = cw_c // _PH
        for ph in range(_PH):
            lo = ph * cwp
            pltpu.sync_copy(src_hbm.at[w, pl.ds(lo, cwp)],
                            src_v.at[pl.ds(0, cwp)])
            pltpu.sync_copy(dst_hbm.at[w, pl.ds(lo, cwp)],
                            dst_v.at[pl.ds(0, cwp)])

            def body(j, carry):
                pltpu.async_copy(xs_hbm.at[src_v.at[j]], g0, s0).wait()
                pltpu.sync_copy(g0, acc.at[dst_v.at[j]], add=True)
                return carry

            lax.fori_loop(0, cwp, body, 0)

    @pl.when(c == 0)
    def _():
        run(s, cw0)

    @pl.when(c == 1)
    def _():
        run(_NS + s, cw1)

    plsc.subcore_barrier()
    for k in range(rows // _CH):
        pltpu.sync_copy(acc.at[pl.ds(base + k * _CH, _CH)], g0)
        pltpu.sync_copy(g0, out_hbm.at[c, pl.ds(base + k * _CH, _CH)])


def _tc_score_body(xpad_ref, pcol_ref, pn_ref, s_ref):
    pnorm = pn_ref[0, 0]
    xb = xpad_ref[...].astype(jnp.bfloat16)
    pb = pcol_ref[...].astype(jnp.bfloat16)
    raw = lax.dot_general(xb, pb, (((1,), (0,)), ((), ())),
                          preferred_element_type=jnp.float32)
    s_ref[...] = raw / pnorm


def _tc_dense_body(n, rf, x_ref, sc_ref, wih_ref, whh_ref, bih_ref,
                   bhh_ref, w0_ref, xw_ref, s_ref, xt_ref):
    sc = sc_ref[...]
    row_i = lax.broadcasted_iota(jnp.int32, (rf, _CH), 0)
    col_i = lax.broadcasted_iota(jnp.int32, (rf, _CH), 1)
    flat = row_i * _CH + col_i
    neg = jnp.float32(-jnp.inf)
    s_ref[...] = jnp.where(flat < n, sc, neg)

    def pick(k, carry):
        sv = s_ref[...]
        m = jnp.max(sv)
        idx = jnp.min(jnp.where(sv == m, flat, jnp.int32(2**30)))
        row = x_ref[pl.ds(idx, 1), :]
        xt_ref[pl.ds(k, 1), :] = row * jnp.tanh(m)
        s_ref[...] = jnp.where(flat == idx, neg, sv)
        return carry

    lax.fori_loop(0, 128, pick, 0)

    xt = xt_ref[...]
    dn = (((1,), (1,)), ((), ()))
    gi = lax.dot_general(xt, wih_ref[...], dn, precision=_HI) + bih_ref[...]
    gh = lax.dot_general(w0_ref[...], whh_ref[...], dn, precision=_HI) + bhh_ref[...]
    d = xt.shape[1]
    i_r, i_z, i_n = gi[:, :d], gi[:, d:2 * d], gi[:, 2 * d:]
    h_r, h_z, h_n = gh[:, :d], gh[:, d:2 * d], gh[:, 2 * d:]
    r = jax.nn.sigmoid(i_r + h_r)
    z = jax.nn.sigmoid(i_z + h_z)
    nn = jnp.tanh(i_n + r * h_n)
    w = (1.0 - z) * nn + z * w0_ref[...]
    xw_ref[...] = lax.dot_general(x_ref[...], w, (((1,), (0,)), ((), ())),
                                  precision=_HI)


def _tc_scale_body(xw_ref, h0_ref, h1_ref, xs_ref, dis_ref):
    deg = 1.0 + h0_ref[...] + h1_ref[...]
    dis = 1.0 / jnp.sqrt(deg)
    dis_ref[...] = dis
    xs_ref[...] = xw_ref[...] * dis


def _tc_final_body(nb, s2_ref, xs_ref, dis_ref, batch_ref, doc_ref, wdoc_ref,
                   bdoc_ref, bng_ref, bnb_ref, bnm_ref, bnv_ref, wfuse_ref,
                   bfuse_ref, whead_ref, bhead_ref, head_ref):
    s2 = s2_ref[...]
    h = dis_ref[...] * (s2[0] + s2[1] + xs_ref[...])
    bids = lax.broadcasted_iota(jnp.int32, (nb, h.shape[0]), 0)
    mask = (batch_ref[...] == bids).astype(jnp.float32)
    dn = (((1,), (0,)), ((), ()))
    ssum = lax.dot_general(mask, h, dn, precision=_HI)
    cnt = jnp.sum(mask, axis=1, keepdims=True)
    pooled = ssum / jnp.maximum(cnt, 1.0)
    dnt = (((1,), (1,)), ((), ()))
    demb = jnp.maximum(
        lax.dot_general(doc_ref[...], wdoc_ref[...], dnt, precision=_HI)
        + bdoc_ref[...], 0.0)
    z = jnp.concatenate([pooled, demb], axis=1)
    z = (z - bnm_ref[...]) / jnp.sqrt(bnv_ref[...] + 1e-5) * bng_ref[...] \
        + bnb_ref[...]
    z2 = jnp.maximum(
        lax.dot_general(z, wfuse_ref[...], dnt, precision=_HI)
        + bfuse_ref[...], 0.0)
    head_ref[...] = lax.dot_general(z2, whead_ref[...], dnt, precision=_HI) \
        + bhead_ref[...]


def kernel(x, doc_features, p, W_ih, W_hh, b_ih, b_hh, W0, W_doc, b_doc,
           bn_gamma, bn_beta, W_fuse, b_fuse, W_task, b_task, W_time, b_time,
           edge_index, batch, bn_mean, bn_var):
    n, d = x.shape
    nb = doc_features.shape[0]
    e = edge_index.shape[1]
    f32 = jnp.float32

    np_ = _rup(n + 1, _NS * _CH)
    ep = _rup(e, _NW * _CH * _PH * 8)
    cwh = ep // (_NW * _CH)
    rf = _rup(n, _CH) // _CH
    total_ch = 2 * cwh
    cw0 = int(total_ch * _SPLIT_FRAC0 / 16) * 16
    cw1 = total_ch - cw0
    cwmax = max(cw0, cw1)

    srcf = jnp.concatenate(
        [edge_index[0], jnp.full((ep - e,), 0, edge_index.dtype)])
    pad_dst = (n + jnp.arange(ep - e, dtype=edge_index.dtype) % (np_ - n))
    dstf = jnp.concatenate([edge_index[1], pad_dst])
    dst_h = dstf.reshape(_NW, cwh, _CH)

    def _asym(flat):
        a0 = flat[:_NS * cw0 * _CH].reshape(_NS, cw0, _CH)
        a1 = flat[_NS * cw0 * _CH:].reshape(_NS, cw1, _CH)
        a0 = jnp.pad(a0, ((0, 0), (0, cwmax - cw0), (0, 0)))
        a1 = jnp.pad(a1, ((0, 0), (0, cwmax - cw1), (0, 0)))
        return jnp.concatenate([a0, a1], axis=0)

    src = _asym(srcf)
    dst = _asym(dstf)
    xpad = jnp.pad(x, ((0, rf * _CH - n), (0, 0)))
    bih2 = b_ih.reshape(1, -1)
    bhh2 = b_hh.reshape(1, -1)
    batchp = jnp.concatenate(
        [batch, jnp.full((np_ - n,), nb, batch.dtype)]).reshape(1, np_)

    mesh = plsc.VectorSubcoreMesh(core_axis_name="c", subcore_axis_name="s",
                                  num_cores=_NC, num_subcores=_NS)

    hist = pl.kernel(
        functools.partial(_sc_hist_body, np_, cwh),
        out_type=jax.ShapeDtypeStruct((_NC, np_, _DEGW), f32),
        mesh=mesh,
        scratch_types=[
            pltpu.VMEM((cwh, _CH), jnp.int32),
            pltpu.VMEM((_CH, _DEGW), f32),
            pltpu.VMEM_SHARED((np_, _DEGW), f32),
        ],
        name="sc_deg_hist",
    )
    deg2 = hist(dst_h)

    score = pl.pallas_call(
        _tc_score_body,
        out_shape=jax.ShapeDtypeStruct((rf * _CH, 1), f32),
        name="tc_score",
    )
    pn = (jnp.linalg.norm(p) + 1e-16).reshape(1, 1)
    sc_fold = score(xpad, p.reshape(d, 1), pn).reshape(rf, _CH)

    dense = pl.pallas_call(
        functools.partial(_tc_dense_body, n, rf),
        out_shape=jax.ShapeDtypeStruct((n, d), f32),
        scratch_shapes=[
            pltpu.VMEM((rf, _CH), f32),
            pltpu.VMEM((d, d), f32),
        ],
        name="tc_dense",
    )
    xw = dense(x, sc_fold, W_ih, W_hh, bih2, bhh2, W0)

    scale = pl.pallas_call(
        _tc_scale_body,
        out_shape=(jax.ShapeDtypeStruct((n, d), f32),
                   jax.ShapeDtypeStruct((n, 1), f32)),
        name="tc_scale",
    )
    xs, dis = scale(xw, deg2[0, :n, 0:1], deg2[1, :n, 0:1])

    xs_p = jnp.pad(xs, ((0, np_ - n), (0, 0)))
    dis_p = jnp.pad(dis, ((0, np_ - n), (0, 0)))

    scat = pl.kernel(
        functools.partial(_sc_scatter_body, np_, cw0, cw1, d),
        out_type=jax.ShapeDtypeStruct((_NC, np_, d), f32),
        mesh=mesh,
        scratch_types=[
            pltpu.VMEM((cwmax // _PH, _CH), jnp.int32),
            pltpu.VMEM((cwmax // _PH, _CH), jnp.int32),
            pltpu.VMEM((_CH, d), f32),
            pltpu.VMEM_SHARED((np_, d), f32),
            pltpu.SemaphoreType.DMA,
        ],
        name="sc_edge_scatter",
    )
    s2 = scat(xs_p, src, dst)

    n_task = W_task.shape[0]
    n_time = W_time.shape[0]
    w_head = jnp.concatenate([W_task, W_time], axis=0)
    b_head = jnp.concatenate([b_task, b_time]).reshape(1, -1)
    final = pl.pallas_call(
        functools.partial(_tc_final_body, nb),
        out_shape=jax.ShapeDtypeStruct((nb, n_task + n_time), f32),
        name="tc_final",
    )
    heads = final(s2, xs_p, dis_p, batchp, doc_features, W_doc,
                  b_doc.reshape(1, -1), bn_gamma.reshape(1, -1),
                  bn_beta.reshape(1, -1), bn_mean.reshape(1, -1),
                  bn_var.reshape(1, -1), W_fuse, b_fuse.reshape(1, -1),
                  w_head, b_head)
    return (heads[:, :n_task], heads[:, n_task:])


# speedup vs baseline: 1.0521x; 1.0001x over previous
import functools

import jax
import jax.numpy as jnp
from jax import lax
from jax.experimental import pallas as pl
from jax.experimental.pallas import tpu as pltpu
from jax.experimental.pallas import tpu_sc as plsc

_NC = 2
_NS = 16
_NW = _NC * _NS
_CH = 128
_DEGW = 128
_PH = 1
_SPLIT_FRAC0 = 0.5

_HI = lax.Precision.HIGHEST


def _rup(a, b):
    return (a + b - 1) // b * b


def _sc_hist_body(np_, cw, dst_hbm, out_hbm, idx_v, buf, acc):
    c = lax.axis_index("c")
    s = lax.axis_index("s")
    w = c * _NS + s
    rows = np_ // _NS
    base = s * rows

    def fill0(i, carry):
        for t in range(_DEGW // 16):
            buf[i, pl.ds(t * 16, 16)] = jnp.zeros((16,), jnp.float32)
        return carry

    lax.fori_loop(0, _CH, fill0, 0)
    for k in range(rows // _CH):
        pltpu.sync_copy(buf, acc.at[pl.ds(base + k * _CH, _CH)])
    pltpu.sync_copy(dst_hbm.at[w], idx_v)

    def fill1(i, carry):
        for t in range(_DEGW // 16):
            buf[i, pl.ds(t * 16, 16)] = jnp.ones((16,), jnp.float32)
        return carry

    lax.fori_loop(0, _CH, fill1, 0)
    plsc.subcore_barrier()

    def scat(j, carry):
        pltpu.sync_copy(buf, acc.at[idx_v.at[j]], add=True)
        return carry

    lax.fori_loop(0, cw, scat, 0)
    plsc.subcore_barrier()
    for k in range(rows // _CH):
        pltpu.sync_copy(acc.at[pl.ds(base + k * _CH, _CH)], buf)
        pltpu.sync_copy(buf, out_hbm.at[c, pl.ds(base + k * _CH, _CH)])


def _sc_scatter_body(np_, cw0, cw1, d, xs_hbm, src_hbm, dst_hbm, out_hbm,
                     src_v, dst_v, g0, acc, s0):
    c = lax.axis_index("c")
    s = lax.axis_index("s")
    rows = np_ // _NS
    base = s * rows

    def zero(i, carry):
        for t in range(d // 16):
            g0[i, pl.ds(t * 16, 16)] = jnp.zeros((16,), jnp.float32)
        return carry

    lax.fori_loop(0, _CH, zero, 0)
    for k in range(rows // _CH):
        pltpu.sync_copy(g0, acc.at[pl.ds(base + k * _CH, _CH)])
    plsc.subcore_barrier()

    w = c * _NS + s
    pltpu.sync_copy(src_hbm.at[w], src_v)
    pltpu.sync_copy(dst_hbm.at[w], dst_v)

    def body(j, carry):
        pltpu.async_copy(xs_hbm.at[src_v.at[j]], g0, s0).wait()
        pltpu.sync_copy(g0, acc.at[dst_v.at[j]], add=True)
        return carry

    lax.fori_loop(0, cw0, body, 0)
    plsc.subcore_barrier()
    for k in range(rows // _CH):
        pltpu.sync_copy(acc.at[pl.ds(base + k * _CH, _CH)], g0)
        pltpu.sync_copy(g0, out_hbm.at[c, pl.ds(base + k * _CH, _CH)])


def _tc_score_body(xpad_ref, pcol_ref, pn_ref, s_ref):
    pnorm = pn_ref[0, 0]
    xb = xpad_ref[...].astype(jnp.bfloat16)
    pb = pcol_ref[...].astype(jnp.bfloat16)
    raw = lax.dot_general(xb, pb, (((1,), (0,)), ((), ())),
                          preferred_element_type=jnp.float32)
    s_ref[...] = raw / pnorm


def _tc_dense_body(n, rf, x_ref, sc_ref, wih_ref, whh_ref, bih_ref,
                   bhh_ref, w0_ref, xw_ref, s_ref, xt_ref):
    sc = sc_ref[...]
    row_i = lax.broadcasted_iota(jnp.int32, (rf, _CH), 0)
    col_i = lax.broadcasted_iota(jnp.int32, (rf, _CH), 1)
    flat = row_i * _CH + col_i
    neg = jnp.float32(-jnp.inf)
    s_ref[...] = jnp.where(flat < n, sc, neg)

    def pick(k, carry):
        sv = s_ref[...]
        m = jnp.max(sv)
        idx = jnp.min(jnp.where(sv == m, flat, jnp.int32(2**30)))
        row = x_ref[pl.ds(idx, 1), :]
        xt_ref[pl.ds(k, 1), :] = row * jnp.tanh(m)
        s_ref[...] = jnp.where(flat == idx, neg, sv)
        return carry

    lax.fori_loop(0, 128, pick, 0)

    xt = xt_ref[...]
    dn = (((1,), (1,)), ((), ()))
    gi = lax.dot_general(xt, wih_ref[...], dn, precision=_HI) + bih_ref[...]
    gh = lax.dot_general(w0_ref[...], whh_ref[...], dn, precision=_HI) + bhh_ref[...]
    d = xt.shape[1]
    i_r, i_z, i_n = gi[:, :d], gi[:, d:2 * d], gi[:, 2 * d:]
    h_r, h_z, h_n = gh[:, :d], gh[:, d:2 * d], gh[:, 2 * d:]
    r = jax.nn.sigmoid(i_r + h_r)
    z = jax.nn.sigmoid(i_z + h_z)
    nn = jnp.tanh(i_n + r * h_n)
    w = (1.0 - z) * nn + z * w0_ref[...]
    xw_ref[...] = lax.dot_general(x_ref[...], w, (((1,), (0,)), ((), ())),
                                  precision=_HI)


def _tc_scale_body(xw_ref, h0_ref, h1_ref, xs_ref, dis_ref):
    deg = 1.0 + h0_ref[...] + h1_ref[...]
    dis = 1.0 / jnp.sqrt(deg)
    dis_ref[...] = dis
    xs_ref[...] = xw_ref[...] * dis


def _tc_final_body(nb, s2_ref, xs_ref, dis_ref, batch_ref, doc_ref, wdoc_ref,
                   bdoc_ref, bng_ref, bnb_ref, bnm_ref, bnv_ref, wfuse_ref,
                   bfuse_ref, whead_ref, bhead_ref, head_ref):
    s2 = s2_ref[...]
    h = dis_ref[...] * (s2[0] + s2[1] + xs_ref[...])
    bids = lax.broadcasted_iota(jnp.int32, (nb, h.shape[0]), 0)
    mask = (batch_ref[...] == bids).astype(jnp.float32)
    dn = (((1,), (0,)), ((), ()))
    ssum = lax.dot_general(mask, h, dn, precision=_HI)
    cnt = jnp.sum(mask, axis=1, keepdims=True)
    pooled = ssum / jnp.maximum(cnt, 1.0)
    dnt = (((1,), (1,)), ((), ()))
    demb = jnp.maximum(
        lax.dot_general(doc_ref[...], wdoc_ref[...], dnt, precision=_HI)
        + bdoc_ref[...], 0.0)
    z = jnp.concatenate([pooled, demb], axis=1)
    z = (z - bnm_ref[...]) / jnp.sqrt(bnv_ref[...] + 1e-5) * bng_ref[...] \
        + bnb_ref[...]
    z2 = jnp.maximum(
        lax.dot_general(z, wfuse_ref[...], dnt, precision=_HI)
        + bfuse_ref[...], 0.0)
    head_ref[...] = lax.dot_general(z2, whead_ref[...], dnt, precision=_HI) \
        + bhead_ref[...]


def kernel(x, doc_features, p, W_ih, W_hh, b_ih, b_hh, W0, W_doc, b_doc,
           bn_gamma, bn_beta, W_fuse, b_fuse, W_task, b_task, W_time, b_time,
           edge_index, batch, bn_mean, bn_var):
    n, d = x.shape
    nb = doc_features.shape[0]
    e = edge_index.shape[1]
    f32 = jnp.float32

    np_ = _rup(n + 1, _NS * _CH)
    ep = _rup(e, _NW * _CH * _PH * 8)
    cwh = ep // (_NW * _CH)
    rf = _rup(n, _CH) // _CH
    total_ch = 2 * cwh
    cw0 = int(total_ch * _SPLIT_FRAC0 / 16) * 16
    cw1 = total_ch - cw0
    cwmax = max(cw0, cw1)

    srcf = jnp.concatenate(
        [edge_index[0], jnp.full((ep - e,), 0, edge_index.dtype)])
    pad_dst = (n + jnp.arange(ep - e, dtype=edge_index.dtype) % (np_ - n))
    dstf = jnp.concatenate([edge_index[1], pad_dst])
    dst_h = dstf.reshape(_NW, cwh, _CH)

    def _asym(flat):
        a0 = flat[:_NS * cw0 * _CH].reshape(_NS, cw0, _CH)
        a1 = flat[_NS * cw0 * _CH:].reshape(_NS, cw1, _CH)
        a0 = jnp.pad(a0, ((0, 0), (0, cwmax - cw0), (0, 0)))
        a1 = jnp.pad(a1, ((0, 0), (0, cwmax - cw1), (0, 0)))
        return jnp.concatenate([a0, a1], axis=0)

    src = _asym(srcf)
    dst = _asym(dstf)
    xpad = jnp.pad(x, ((0, rf * _CH - n), (0, 0)))
    bih2 = b_ih.reshape(1, -1)
    bhh2 = b_hh.reshape(1, -1)
    batchp = jnp.concatenate(
        [batch, jnp.full((np_ - n,), nb, batch.dtype)]).reshape(1, np_)

    mesh = plsc.VectorSubcoreMesh(core_axis_name="c", subcore_axis_name="s",
                                  num_cores=_NC, num_subcores=_NS)

    hist = pl.kernel(
        functools.partial(_sc_hist_body, np_, cwh),
        out_type=jax.ShapeDtypeStruct((_NC, np_, _DEGW), f32),
        mesh=mesh,
        scratch_types=[
            pltpu.VMEM((cwh, _CH), jnp.int32),
            pltpu.VMEM((_CH, _DEGW), f32),
            pltpu.VMEM_SHARED((np_, _DEGW), f32),
        ],
        name="sc_deg_hist",
    )
    deg2 = hist(dst_h)

    score = pl.pallas_call(
        _tc_score_body,
        out_shape=jax.ShapeDtypeStruct((rf * _CH, 1), f32),
        name="tc_score",
    )
    pn = (jnp.linalg.norm(p) + 1e-16).reshape(1, 1)
    sc_fold = score(xpad, p.reshape(d, 1), pn).reshape(rf, _CH)

    dense = pl.pallas_call(
        functools.partial(_tc_dense_body, n, rf),
        out_shape=jax.ShapeDtypeStruct((n, d), f32),
        scratch_shapes=[
            pltpu.VMEM((rf, _CH), f32),
            pltpu.VMEM((d, d), f32),
        ],
        name="tc_dense",
    )
    xw = dense(x, sc_fold, W_ih, W_hh, bih2, bhh2, W0)

    scale = pl.pallas_call(
        _tc_scale_body,
        out_shape=(jax.ShapeDtypeStruct((n, d), f32),
                   jax.ShapeDtypeStruct((n, 1), f32)),
        name="tc_scale",
    )
    xs, dis = scale(xw, deg2[0, :n, 0:1], deg2[1, :n, 0:1])

    xs_p = jnp.pad(xs, ((0, np_ - n), (0, 0)))
    dis_p = jnp.pad(dis, ((0, np_ - n), (0, 0)))

    scat = pl.kernel(
        functools.partial(_sc_scatter_body, np_, cw0, cw1, d),
        out_type=jax.ShapeDtypeStruct((_NC, np_, d), f32),
        mesh=mesh,
        scratch_types=[
            pltpu.VMEM((cwmax // _PH, _CH), jnp.int32),
            pltpu.VMEM((cwmax // _PH, _CH), jnp.int32),
            pltpu.VMEM((_CH, d), f32),
            pltpu.VMEM_SHARED((np_, d), f32),
            pltpu.SemaphoreType.DMA,
        ],
        name="sc_edge_scatter",
    )
    s2 = scat(xs_p, src, dst)

    n_task = W_task.shape[0]
    n_time = W_time.shape[0]
    w_head = jnp.concatenate([W_task, W_time], axis=0)
    b_head = jnp.concatenate([b_task, b_time]).reshape(1, -1)
    final = pl.pallas_call(
        functools.partial(_tc_final_body, nb),
        out_shape=jax.ShapeDtypeStruct((nb, n_task + n_time), f32),
        name="tc_final",
    )
    heads = final(s2, xs_p, dis_p, batchp, doc_features, W_doc,
                  b_doc.reshape(1, -1), bn_gamma.reshape(1, -1),
                  bn_beta.reshape(1, -1), bn_mean.reshape(1, -1),
                  bn_var.reshape(1, -1), W_fuse, b_fuse.reshape(1, -1),
                  w_head, b_head)
    return (heads[:, :n_task], heads[:, n_task:])


# speedup vs baseline: 1.0523x; 1.0002x over previous
import functools

import jax
import jax.numpy as jnp
from jax import lax
from jax.experimental import pallas as pl
from jax.experimental.pallas import tpu as pltpu
from jax.experimental.pallas import tpu_sc as plsc

_NC = 2
_NS = 16
_NW = _NC * _NS
_CH = 128
_DEGW = 128
_PH = 1
_SPLIT_FRAC0 = 0.5

_HI = lax.Precision.HIGHEST


def _rup(a, b):
    return (a + b - 1) // b * b


def _sc_hist_body(np_, cw, dst_hbm, out_hbm, idx_v, buf, acc):
    c = lax.axis_index("c")
    s = lax.axis_index("s")
    w = c * _NS + s
    rows = np_ // _NS
    base = s * rows

    def fill0(i, carry):
        for t in range(_DEGW // 16):
            buf[i, pl.ds(t * 16, 16)] = jnp.zeros((16,), jnp.float32)
        return carry

    lax.fori_loop(0, _CH, fill0, 0)
    for k in range(rows // _CH):
        pltpu.sync_copy(buf, acc.at[pl.ds(base + k * _CH, _CH)])
    pltpu.sync_copy(dst_hbm.at[w], idx_v)

    def fill1(i, carry):
        for t in range(_DEGW // 16):
            buf[i, pl.ds(t * 16, 16)] = jnp.ones((16,), jnp.float32)
        return carry

    lax.fori_loop(0, _CH, fill1, 0)
    plsc.subcore_barrier()

    def scat(j, carry):
        pltpu.sync_copy(buf, acc.at[idx_v.at[j]], add=True)
        return carry

    lax.fori_loop(0, cw, scat, 0)
    plsc.subcore_barrier()
    for k in range(rows // _CH):
        pltpu.sync_copy(acc.at[pl.ds(base + k * _CH, _CH)], buf)
        pltpu.sync_copy(buf, out_hbm.at[c, pl.ds(base + k * _CH, _CH)])


def _sc_scatter_body(np_, cw0, cw1, d, xs_hbm, src_hbm, dst_hbm, out_hbm,
                     src_v, dst_v, g0, acc, s0):
    c = lax.axis_index("c")
    s = lax.axis_index("s")
    rows = np_ // _NS
    base = s * rows

    def zero(i, carry):
        for t in range(d // 16):
            g0[i, pl.ds(t * 16, 16)] = jnp.zeros((16,), jnp.float32)
        return carry

    lax.fori_loop(0, _CH, zero, 0)
    for k in range(rows // _CH):
        pltpu.sync_copy(g0, acc.at[pl.ds(base + k * _CH, _CH)])
    plsc.subcore_barrier()

    w = c * _NS + s
    pltpu.sync_copy(src_hbm.at[w], src_v)
    pltpu.sync_copy(dst_hbm.at[w], dst_v)

    def body(j, carry):
        pltpu.async_copy(xs_hbm.at[src_v.at[j]], g0, s0).wait()
        pltpu.sync_copy(g0, acc.at[dst_v.at[j]], add=True)
        return carry

    lax.fori_loop(0, cw0, body, 0)
    plsc.subcore_barrier()
    for k in range(rows // _CH):
        pltpu.sync_copy(acc.at[pl.ds(base + k * _CH, _CH)], g0)
        pltpu.sync_copy(g0, out_hbm.at[c, pl.ds(base + k * _CH, _CH)])


def _tc_score_body(xpad_ref, pcol_ref, pn_ref, s_ref):
    pnorm = pn_ref[0, 0]
    xb = xpad_ref[...].astype(jnp.bfloat16)
    pb = pcol_ref[...].astype(jnp.bfloat16)
    raw = lax.dot_general(xb, pb, (((1,), (0,)), ((), ())),
                          preferred_element_type=jnp.float32)
    s_ref[...] = raw / pnorm


def _tc_dense_body(n, rf, x_ref, sc_ref, wih_ref, whh_ref, bih_ref,
                   bhh_ref, w0_ref, xw_ref, s_ref, xt_ref):
    sc = sc_ref[...]
    row_i = lax.broadcasted_iota(jnp.int32, (rf, _CH), 0)
    col_i = lax.broadcasted_iota(jnp.int32, (rf, _CH), 1)
    flat = row_i * _CH + col_i
    neg = jnp.float32(-jnp.inf)
    s_ref[...] = jnp.where(flat < n, sc, neg)

    def pick(k, carry):
        sv = s_ref[...]
        m = jnp.max(sv)
        idx = jnp.min(jnp.where(sv == m, flat, jnp.int32(2**30)))
        row = x_ref[pl.ds(idx, 1), :]
        xt_ref[pl.ds(k, 1), :] = row * jnp.tanh(m)
        s_ref[...] = jnp.where(flat == idx, neg, sv)
        return carry

    lax.fori_loop(0, 128, pick, 0)

    xt = xt_ref[...]
    dn = (((1,), (1,)), ((), ()))
    gi = lax.dot_general(xt, wih_ref[...], dn, precision=_HI) + bih_ref[...]
    gh = lax.dot_general(w0_ref[...], whh_ref[...], dn, precision=_HI) + bhh_ref[...]
    d = xt.shape[1]
    i_r, i_z, i_n = gi[:, :d], gi[:, d:2 * d], gi[:, 2 * d:]
    h_r, h_z, h_n = gh[:, :d], gh[:, d:2 * d], gh[:, 2 * d:]
    r = jax.nn.sigmoid(i_r + h_r)
    z = jax.nn.sigmoid(i_z + h_z)
    nn = jnp.tanh(i_n + r * h_n)
    w = (1.0 - z) * nn + z * w0_ref[...]
    xw_ref[...] = lax.dot_general(x_ref[...], w, (((1,), (0,)), ((), ())),
                                  precision=_HI)


def _tc_scale_body(xw_ref, h0_ref, h1_ref, xs_ref, dis_ref):
    deg = 1.0 + h0_ref[...] + h1_ref[...]
    dis = 1.0 / jnp.sqrt(deg)
    dis_ref[...] = dis
    xs_ref[...] = xw_ref[...] * dis


def _tc_final_body(nb, s2_ref, xs_ref, dis_ref, batch_ref, doc_ref, wdoc_ref,
                   bdoc_ref, bng_ref, bnb_ref, bnm_ref, bnv_ref, wfuse_ref,
                   bfuse_ref, whead_ref, bhead_ref, head_ref):
    s2 = s2_ref[...]
    h = dis_ref[...] * (s2[0] + s2[1] + xs_ref[...])
    bids = lax.broadcasted_iota(jnp.int32, (nb, h.shape[0]), 0)
    mask = (batch_ref[...] == bids).astype(jnp.float32)
    dn = (((1,), (0,)), ((), ()))
    ssum = lax.dot_general(mask, h, dn, precision=_HI)
    cnt = jnp.sum(mask, axis=1, keepdims=True)
    pooled = ssum / jnp.maximum(cnt, 1.0)
    dnt = (((1,), (1,)), ((), ()))
    demb = jnp.maximum(
        lax.dot_general(doc_ref[...], wdoc_ref[...], dnt, precision=_HI)
        + bdoc_ref[...], 0.0)
    z = jnp.concatenate([pooled, demb], axis=1)
    z = (z - bnm_ref[...]) / jnp.sqrt(bnv_ref[...] + 1e-5) * bng_ref[...] \
        + bnb_ref[...]
    z2 = jnp.maximum(
        lax.dot_general(z, wfuse_ref[...], dnt, precision=_HI)
        + bfuse_ref[...], 0.0)
    head_ref[...] = lax.dot_general(z2, whead_ref[...], dnt, precision=_HI) \
        + bhead_ref[...]


def kernel(x, doc_features, p, W_ih, W_hh, b_ih, b_hh, W0, W_doc, b_doc,
           bn_gamma, bn_beta, W_fuse, b_fuse, W_task, b_task, W_time, b_time,
           edge_index, batch, bn_mean, bn_var):
    n, d = x.shape
    nb = doc_features.shape[0]
    e = edge_index.shape[1]
    f32 = jnp.float32

    np_ = _rup(n + 1, _NS * _CH)
    ep = _rup(e, _NW * _CH * _PH * 8)
    cwh = ep // (_NW * _CH)
    rf = _rup(n, _CH) // _CH
    total_ch = 2 * cwh
    cw0 = int(total_ch * _SPLIT_FRAC0 / 16) * 16
    cw1 = total_ch - cw0
    cwmax = max(cw0, cw1)

    srcf = jnp.concatenate(
        [edge_index[0], jnp.full((ep - e,), 0, edge_index.dtype)])
    dstf = jnp.concatenate(
        [edge_index[1], jnp.full((ep - e,), n, edge_index.dtype)])
    dst_h = dstf.reshape(_NW, cwh, _CH)

    def _asym(flat):
        a0 = flat[:_NS * cw0 * _CH].reshape(_NS, cw0, _CH)
        a1 = flat[_NS * cw0 * _CH:].reshape(_NS, cw1, _CH)
        a0 = jnp.pad(a0, ((0, 0), (0, cwmax - cw0), (0, 0)))
        a1 = jnp.pad(a1, ((0, 0), (0, cwmax - cw1), (0, 0)))
        return jnp.concatenate([a0, a1], axis=0)

    src = _asym(srcf)
    dst = _asym(dstf)
    xpad = jnp.pad(x, ((0, rf * _CH - n), (0, 0)))
    bih2 = b_ih.reshape(1, -1)
    bhh2 = b_hh.reshape(1, -1)
    batchp = jnp.concatenate(
        [batch, jnp.full((np_ - n,), nb, batch.dtype)]).reshape(1, np_)

    mesh = plsc.VectorSubcoreMesh(core_axis_name="c", subcore_axis_name="s",
                                  num_cores=_NC, num_subcores=_NS)

    hist = pl.kernel(
        functools.partial(_sc_hist_body, np_, cwh),
        out_type=jax.ShapeDtypeStruct((_NC, np_, _DEGW), f32),
        mesh=mesh,
        scratch_types=[
            pltpu.VMEM((cwh, _CH), jnp.int32),
            pltpu.VMEM((_CH, _DEGW), f32),
            pltpu.VMEM_SHARED((np_, _DEGW), f32),
        ],
        name="sc_deg_hist",
    )
    deg2 = hist(dst_h)

    score = pl.pallas_call(
        _tc_score_body,
        out_shape=jax.ShapeDtypeStruct((rf * _CH, 1), f32),
        name="tc_score",
    )
    pn = (jnp.linalg.norm(p) + 1e-16).reshape(1, 1)
    sc_fold = score(xpad, p.reshape(d, 1), pn).reshape(rf, _CH)

    dense = pl.pallas_call(
        functools.partial(_tc_dense_body, n, rf),
        out_shape=jax.ShapeDtypeStruct((n, d), f32),
        scratch_shapes=[
            pltpu.VMEM((rf, _CH), f32),
            pltpu.VMEM((d, d), f32),
        ],
        name="tc_dense",
    )
    xw = dense(x, sc_fold, W_ih, W_hh, bih2, bhh2, W0)

    scale = pl.pallas_call(
        _tc_scale_body,
        out_shape=(jax.ShapeDtypeStruct((n, d), f32),
                   jax.ShapeDtypeStruct((n, 1), f32)),
        name="tc_scale",
    )
    xs, dis = scale(xw, deg2[0, :n, 0:1], deg2[1, :n, 0:1])

    xs_p = jnp.pad(xs, ((0, np_ - n), (0, 0)))
    dis_p = jnp.pad(dis, ((0, np_ - n), (0, 0)))

    scat = pl.kernel(
        functools.partial(_sc_scatter_body, np_, cw0, cw1, d),
        out_type=jax.ShapeDtypeStruct((_NC, np_, d), f32),
        mesh=mesh,
        scratch_types=[
            pltpu.VMEM((cwmax // _PH, _CH), jnp.int32),
            pltpu.VMEM((cwmax // _PH, _CH), jnp.int32),
            pltpu.VMEM((_CH, d), f32),
            pltpu.VMEM_SHARED((np_, d), f32),
            pltpu.SemaphoreType.DMA,
        ],
        name="sc_edge_scatter",
    )
    s2 = scat(xs_p, src, dst)

    n_task = W_task.shape[0]
    n_time = W_time.shape[0]
    w_head = jnp.concatenate([W_task, W_time], axis=0)
    b_head = jnp.concatenate([b_task, b_time]).reshape(1, -1)
    final = pl.pallas_call(
        functools.partial(_tc_final_body, nb),
        out_shape=jax.ShapeDtypeStruct((nb, n_task + n_time), f32),
        name="tc_final",
    )
    heads = final(s2, xs_p, dis_p, batchp, doc_features, W_doc,
                  b_doc.reshape(1, -1), bn_gamma.reshape(1, -1),
                  bn_beta.reshape(1, -1), bn_mean.reshape(1, -1),
                  bn_var.reshape(1, -1), W_fuse, b_fuse.reshape(1, -1),
                  w_head, b_head)
    return (heads[:, :n_task], heads[:, n_task:])


# speedup vs baseline: 1.3278x; 1.2619x over previous
import functools

import jax
import jax.numpy as jnp
from jax import lax
from jax.experimental import pallas as pl
from jax.experimental.pallas import tpu as pltpu
from jax.experimental.pallas import tpu_sc as plsc

_NC = 2
_NS = 16
_NW = _NC * _NS
_CH = 128
_DEGW = 128
_PH = 1
_SPLIT_FRAC0 = 0.5

_HI = lax.Precision.HIGHEST


def _rup(a, b):
    return (a + b - 1) // b * b


def _sc_hist_body(np_, cw, dst_hbm, out_hbm, idx_v, buf, acc):
    c = lax.axis_index("c")
    s = lax.axis_index("s")
    w = c * _NS + s
    rows = np_ // _NS
    base = s * rows

    def fill0(i, carry):
        for t in range(_DEGW // 16):
            buf[i, pl.ds(t * 16, 16)] = jnp.zeros((16,), jnp.float32)
        return carry

    lax.fori_loop(0, _CH, fill0, 0)
    for k in range(rows // _CH):
        pltpu.sync_copy(buf, acc.at[pl.ds(base + k * _CH, _CH)])
    pltpu.sync_copy(dst_hbm.at[w], idx_v)

    def fill1(i, carry):
        for t in range(_DEGW // 16):
            buf[i, pl.ds(t * 16, 16)] = jnp.ones((16,), jnp.float32)
        return carry

    lax.fori_loop(0, _CH, fill1, 0)
    plsc.subcore_barrier()

    def scat(j, carry):
        pltpu.sync_copy(buf, acc.at[idx_v.at[j]], add=True)
        return carry

    lax.fori_loop(0, cw, scat, 0)
    plsc.subcore_barrier()
    for k in range(rows // _CH):
        pltpu.sync_copy(acc.at[pl.ds(base + k * _CH, _CH)], buf)
        pltpu.sync_copy(buf, out_hbm.at[c, pl.ds(base + k * _CH, _CH)])


def _sc_scatter_body(np_, cw0, d, xs_hbm, src_hbm, dst_hbm, out_hbm,
                     src_v, dst_v, g0, acc, s0):
    c = lax.axis_index("c")
    s = lax.axis_index("s")
    rows = np_ // _NS
    base = s * rows

    def zero(i, carry):
        for t in range(d // 16):
            g0[i, pl.ds(t * 16, 16)] = jnp.zeros((16,), jnp.float32)
        return carry

    lax.fori_loop(0, _CH, zero, 0)
    for k in range(rows // _CH):
        pltpu.sync_copy(g0, acc.at[pl.ds(base + k * _CH, _CH)])
    plsc.subcore_barrier()

    w = c * _NS + s
    pltpu.sync_copy(src_hbm.at[w], src_v)
    pltpu.sync_copy(dst_hbm.at[w], dst_v)

    def body(j, carry):
        pltpu.async_copy(xs_hbm.at[src_v.at[j]], g0, s0).wait()
        pltpu.sync_copy(g0, acc.at[dst_v.at[j]], add=True)
        return carry

    lax.fori_loop(0, cw0, body, 0)
    plsc.subcore_barrier()
    for k in range(rows // _CH):
        pltpu.sync_copy(acc.at[pl.ds(base + k * _CH, _CH)], g0)
        pltpu.sync_copy(g0, out_hbm.at[c, pl.ds(base + k * _CH, _CH)])


def _tc_score_body(xpad_ref, pcol_ref, pn_ref, s_ref):
    pnorm = pn_ref[0, 0]
    xb = xpad_ref[...].astype(jnp.bfloat16)
    pb = pcol_ref[...].astype(jnp.bfloat16)
    raw = lax.dot_general(xb, pb, (((1,), (0,)), ((), ())),
                          preferred_element_type=jnp.float32)
    s_ref[...] = raw / pnorm


def _tc_dense_body(n, rf, x_ref, sc_ref, wih_ref, whh_ref, bih_ref,
                   bhh_ref, w0_ref, xw_ref, s_ref, xt_ref):
    sc = sc_ref[...]
    row_i = lax.broadcasted_iota(jnp.int32, (rf, _CH), 0)
    col_i = lax.broadcasted_iota(jnp.int32, (rf, _CH), 1)
    flat = row_i * _CH + col_i
    neg = jnp.float32(-jnp.inf)
    s_ref[...] = jnp.where(flat < n, sc, neg)

    def pick(k, carry):
        sv = s_ref[...]
        m = jnp.max(sv)
        idx = jnp.min(jnp.where(sv == m, flat, jnp.int32(2**30)))
        row = x_ref[pl.ds(idx, 1), :]
        xt_ref[pl.ds(k, 1), :] = row * jnp.tanh(m)
        s_ref[...] = jnp.where(flat == idx, neg, sv)
        return carry

    lax.fori_loop(0, 128, pick, 0)

    xt = xt_ref[...]
    dn = (((1,), (1,)), ((), ()))
    gi = lax.dot_general(xt, wih_ref[...], dn, precision=_HI) + bih_ref[...]
    gh = lax.dot_general(w0_ref[...], whh_ref[...], dn, precision=_HI) + bhh_ref[...]
    d = xt.shape[1]
    i_r, i_z, i_n = gi[:, :d], gi[:, d:2 * d], gi[:, 2 * d:]
    h_r, h_z, h_n = gh[:, :d], gh[:, d:2 * d], gh[:, 2 * d:]
    r = jax.nn.sigmoid(i_r + h_r)
    z = jax.nn.sigmoid(i_z + h_z)
    nn = jnp.tanh(i_n + r * h_n)
    w = (1.0 - z) * nn + z * w0_ref[...]
    xw_ref[...] = lax.dot_general(x_ref[...], w, (((1,), (0,)), ((), ())),
                                  precision=_HI)


def _tc_scale_body(xw_ref, h0_ref, h1_ref, xs_ref, dis_ref):
    deg = 1.0 + h0_ref[...] + h1_ref[...]
    dis = 1.0 / jnp.sqrt(deg)
    dis_ref[...] = dis
    xs_ref[...] = xw_ref[...] * dis


def _tc_final_body(nb, s2_ref, xs_ref, dis_ref, batch_ref, doc_ref, wdoc_ref,
                   bdoc_ref, bng_ref, bnb_ref, bnm_ref, bnv_ref, wfuse_ref,
                   bfuse_ref, whead_ref, bhead_ref, head_ref):
    s2 = s2_ref[...]
    h = dis_ref[...] * (s2[0] + s2[1] + xs_ref[...])
    bids = lax.broadcasted_iota(jnp.int32, (nb, h.shape[0]), 0)
    mask = (batch_ref[...] == bids).astype(jnp.float32)
    dn = (((1,), (0,)), ((), ()))
    ssum = lax.dot_general(mask, h, dn, precision=_HI)
    cnt = jnp.sum(mask, axis=1, keepdims=True)
    pooled = ssum / jnp.maximum(cnt, 1.0)
    dnt = (((1,), (1,)), ((), ()))
    demb = jnp.maximum(
        lax.dot_general(doc_ref[...], wdoc_ref[...], dnt, precision=_HI)
        + bdoc_ref[...], 0.0)
    z = jnp.concatenate([pooled, demb], axis=1)
    z = (z - bnm_ref[...]) / jnp.sqrt(bnv_ref[...] + 1e-5) * bng_ref[...] \
        + bnb_ref[...]
    z2 = jnp.maximum(
        lax.dot_general(z, wfuse_ref[...], dnt, precision=_HI)
        + bfuse_ref[...], 0.0)
    head_ref[...] = lax.dot_general(z2, whead_ref[...], dnt, precision=_HI) \
        + bhead_ref[...]


def kernel(x, doc_features, p, W_ih, W_hh, b_ih, b_hh, W0, W_doc, b_doc,
           bn_gamma, bn_beta, W_fuse, b_fuse, W_task, b_task, W_time, b_time,
           edge_index, batch, bn_mean, bn_var):
    n, d = x.shape
    nb = doc_features.shape[0]
    e = edge_index.shape[1]
    f32 = jnp.float32

    np_ = _rup(n + 1, _NS * _CH)
    ep = _rup(e, _NW * _CH)
    cwh = ep // (_NW * _CH)
    rf = _rup(n, _CH) // _CH
    cw0 = cwh
    cwmax = cwh

    src = jnp.concatenate(
        [edge_index[0], jnp.full((ep - e,), 0, edge_index.dtype)]
    ).reshape(_NW, cwh, _CH)
    dst = jnp.concatenate(
        [edge_index[1], jnp.full((ep - e,), n, edge_index.dtype)]
    ).reshape(_NW, cwh, _CH)
    dst_h = dst
    xpad = jnp.pad(x, ((0, rf * _CH - n), (0, 0)))
    bih2 = b_ih.reshape(1, -1)
    bhh2 = b_hh.reshape(1, -1)
    batchp = jnp.concatenate(
        [batch, jnp.full((np_ - n,), nb, batch.dtype)]).reshape(1, np_)

    mesh = plsc.VectorSubcoreMesh(core_axis_name="c", subcore_axis_name="s",
                                  num_cores=_NC, num_subcores=_NS)

    hist = pl.kernel(
        functools.partial(_sc_hist_body, np_, cwh),
        out_type=jax.ShapeDtypeStruct((_NC, np_, _DEGW), f32),
        mesh=mesh,
        scratch_types=[
            pltpu.VMEM((cwh, _CH), jnp.int32),
            pltpu.VMEM((_CH, _DEGW), f32),
            pltpu.VMEM_SHARED((np_, _DEGW), f32),
        ],
        name="sc_deg_hist",
    )
    deg2 = hist(dst_h)

    score = pl.pallas_call(
        _tc_score_body,
        out_shape=jax.ShapeDtypeStruct((rf * _CH, 1), f32),
        name="tc_score",
    )
    pn = (jnp.linalg.norm(p) + 1e-16).reshape(1, 1)
    sc_fold = score(xpad, p.reshape(d, 1), pn).reshape(rf, _CH)

    dense = pl.pallas_call(
        functools.partial(_tc_dense_body, n, rf),
        out_shape=jax.ShapeDtypeStruct((n, d), f32),
        scratch_shapes=[
            pltpu.VMEM((rf, _CH), f32),
            pltpu.VMEM((d, d), f32),
        ],
        name="tc_dense",
    )
    xw = dense(x, sc_fold, W_ih, W_hh, bih2, bhh2, W0)

    scale = pl.pallas_call(
        _tc_scale_body,
        out_shape=(jax.ShapeDtypeStruct((n, d), f32),
                   jax.ShapeDtypeStruct((n, 1), f32)),
        name="tc_scale",
    )
    xs, dis = scale(xw, deg2[0, :n, 0:1], deg2[1, :n, 0:1])

    xs_p = jnp.pad(xs, ((0, np_ - n), (0, 0)))
    dis_p = jnp.pad(dis, ((0, np_ - n), (0, 0)))

    scat = pl.kernel(
        functools.partial(_sc_scatter_body, np_, cw0, d),
        out_type=jax.ShapeDtypeStruct((_NC, np_, d), f32),
        mesh=mesh,
        scratch_types=[
            pltpu.VMEM((cwmax, _CH), jnp.int32),
            pltpu.VMEM((cwmax, _CH), jnp.int32),
            pltpu.VMEM((_CH, d), f32),
            pltpu.VMEM_SHARED((np_, d), f32),
            pltpu.SemaphoreType.DMA,
        ],
        name="sc_edge_scatter",
    )
    s2 = scat(xs_p, src, dst)

    n_task = W_task.shape[0]
    n_time = W_time.shape[0]
    w_head = jnp.concatenate([W_task, W_time], axis=0)
    b_head = jnp.concatenate([b_task, b_time]).reshape(1, -1)
    final = pl.pallas_call(
        functools.partial(_tc_final_body, nb),
        out_shape=jax.ShapeDtypeStruct((nb, n_task + n_time), f32),
        name="tc_final",
    )
    heads = final(s2, xs_p, dis_p, batchp, doc_features, W_doc,
                  b_doc.reshape(1, -1), bn_gamma.reshape(1, -1),
                  bn_beta.reshape(1, -1), bn_mean.reshape(1, -1),
                  bn_var.reshape(1, -1), W_fuse, b_fuse.reshape(1, -1),
                  w_head, b_head)
    return (heads[:, :n_task], heads[:, n_task:])


# speedup vs baseline: 1.3375x; 1.0072x over previous
import functools

import jax
import jax.numpy as jnp
from jax import lax
from jax.experimental import pallas as pl
from jax.experimental.pallas import tpu as pltpu
from jax.experimental.pallas import tpu_sc as plsc

_NC = 2
_NS = 16
_NW = _NC * _NS
_CH = 128
_DEGW = 128
_PH = 1
_SPLIT_FRAC0 = 0.5

_HI = lax.Precision.HIGHEST


def _bfdot(a, b, dnums):
    return lax.dot_general(a.astype(jnp.bfloat16), b.astype(jnp.bfloat16),
                           dnums, preferred_element_type=jnp.float32)


def _rup(a, b):
    return (a + b - 1) // b * b


def _sc_hist_body(np_, cw, dst_hbm, out_hbm, idx_v, buf, acc):
    c = lax.axis_index("c")
    s = lax.axis_index("s")
    w = c * _NS + s
    rows = np_ // _NS
    base = s * rows

    def fill0(i, carry):
        for t in range(_DEGW // 16):
            buf[i, pl.ds(t * 16, 16)] = jnp.zeros((16,), jnp.float32)
        return carry

    lax.fori_loop(0, _CH, fill0, 0)
    for k in range(rows // _CH):
        pltpu.sync_copy(buf, acc.at[pl.ds(base + k * _CH, _CH)])
    pltpu.sync_copy(dst_hbm.at[w], idx_v)

    def fill1(i, carry):
        for t in range(_DEGW // 16):
            buf[i, pl.ds(t * 16, 16)] = jnp.ones((16,), jnp.float32)
        return carry

    lax.fori_loop(0, _CH, fill1, 0)
    plsc.subcore_barrier()

    def scat(j, carry):
        pltpu.sync_copy(buf, acc.at[idx_v.at[j]], add=True)
        return carry

    lax.fori_loop(0, cw, scat, 0)
    plsc.subcore_barrier()
    for k in range(rows // _CH):
        pltpu.sync_copy(acc.at[pl.ds(base + k * _CH, _CH)], buf)
        pltpu.sync_copy(buf, out_hbm.at[c, pl.ds(base + k * _CH, _CH)])


def _sc_scatter_body(np_, cw0, d, xs_hbm, src_hbm, dst_hbm, out_hbm,
                     src_v, dst_v, g0, acc, s0):
    c = lax.axis_index("c")
    s = lax.axis_index("s")
    rows = np_ // _NS
    base = s * rows

    def zero(i, carry):
        for t in range(d // 16):
            g0[i, pl.ds(t * 16, 16)] = jnp.zeros((16,), jnp.float32)
        return carry

    lax.fori_loop(0, _CH, zero, 0)
    for k in range(rows // _CH):
        pltpu.sync_copy(g0, acc.at[pl.ds(base + k * _CH, _CH)])
    plsc.subcore_barrier()

    w = c * _NS + s
    pltpu.sync_copy(src_hbm.at[w], src_v)
    pltpu.sync_copy(dst_hbm.at[w], dst_v)

    def body(j, carry):
        pltpu.async_copy(xs_hbm.at[src_v.at[j]], g0, s0).wait()
        pltpu.sync_copy(g0, acc.at[dst_v.at[j]], add=True)
        return carry

    lax.fori_loop(0, cw0, body, 0)
    plsc.subcore_barrier()
    for k in range(rows // _CH):
        pltpu.sync_copy(acc.at[pl.ds(base + k * _CH, _CH)], g0)
        pltpu.sync_copy(g0, out_hbm.at[c, pl.ds(base + k * _CH, _CH)])


def _tc_score_body(xpad_ref, pcol_ref, pn_ref, s_ref):
    pnorm = pn_ref[0, 0]
    xb = xpad_ref[...].astype(jnp.bfloat16)
    pb = pcol_ref[...].astype(jnp.bfloat16)
    raw = lax.dot_general(xb, pb, (((1,), (0,)), ((), ())),
                          preferred_element_type=jnp.float32)
    s_ref[...] = raw / pnorm


def _tc_dense_body(n, rf, x_ref, sc_ref, wih_ref, whh_ref, bih_ref,
                   bhh_ref, w0_ref, xw_ref, s_ref, xt_ref):
    sc = sc_ref[...]
    row_i = lax.broadcasted_iota(jnp.int32, (rf, _CH), 0)
    col_i = lax.broadcasted_iota(jnp.int32, (rf, _CH), 1)
    flat = row_i * _CH + col_i
    neg = jnp.float32(-jnp.inf)
    s_ref[...] = jnp.where(flat < n, sc, neg)

    def pick(k, carry):
        sv = s_ref[...]
        m = jnp.max(sv)
        idx = jnp.min(jnp.where(sv == m, flat, jnp.int32(2**30)))
        row = x_ref[pl.ds(idx, 1), :]
        xt_ref[pl.ds(k, 1), :] = row * jnp.tanh(m)
        s_ref[...] = jnp.where(flat == idx, neg, sv)
        return carry

    lax.fori_loop(0, 128, pick, 0)

    xt = xt_ref[...]
    dn = (((1,), (1,)), ((), ()))
    gi = _bfdot(xt, wih_ref[...], dn) + bih_ref[...]
    gh = _bfdot(w0_ref[...], whh_ref[...], dn) + bhh_ref[...]
    d = xt.shape[1]
    i_r, i_z, i_n = gi[:, :d], gi[:, d:2 * d], gi[:, 2 * d:]
    h_r, h_z, h_n = gh[:, :d], gh[:, d:2 * d], gh[:, 2 * d:]
    r = jax.nn.sigmoid(i_r + h_r)
    z = jax.nn.sigmoid(i_z + h_z)
    nn = jnp.tanh(i_n + r * h_n)
    w = (1.0 - z) * nn + z * w0_ref[...]
    xw_ref[...] = _bfdot(x_ref[...], w, (((1,), (0,)), ((), ())))


def _tc_scale_body(xw_ref, h0_ref, h1_ref, xs_ref, dis_ref):
    deg = 1.0 + h0_ref[...] + h1_ref[...]
    dis = 1.0 / jnp.sqrt(deg)
    dis_ref[...] = dis
    xs_ref[...] = xw_ref[...] * dis


def _tc_final_body(nb, s2_ref, xs_ref, dis_ref, batch_ref, doc_ref, wdoc_ref,
                   bdoc_ref, bng_ref, bnb_ref, bnm_ref, bnv_ref, wfuse_ref,
                   bfuse_ref, whead_ref, bhead_ref, head_ref):
    s2 = s2_ref[...]
    h = dis_ref[...] * (s2[0] + s2[1] + xs_ref[...])
    bids = lax.broadcasted_iota(jnp.int32, (nb, h.shape[0]), 0)
    mask = (batch_ref[...] == bids).astype(jnp.float32)
    dn = (((1,), (0,)), ((), ()))
    ssum = lax.dot_general(mask, h, dn, precision=_HI)
    cnt = jnp.sum(mask, axis=1, keepdims=True)
    pooled = ssum / jnp.maximum(cnt, 1.0)
    dnt = (((1,), (1,)), ((), ()))
    demb = jnp.maximum(
        _bfdot(doc_ref[...], wdoc_ref[...], dnt) + bdoc_ref[...], 0.0)
    z = jnp.concatenate([pooled, demb], axis=1)
    z = (z - bnm_ref[...]) / jnp.sqrt(bnv_ref[...] + 1e-5) * bng_ref[...] \
        + bnb_ref[...]
    z2 = jnp.maximum(_bfdot(z, wfuse_ref[...], dnt) + bfuse_ref[...], 0.0)
    head_ref[...] = _bfdot(z2, whead_ref[...], dnt) + bhead_ref[...]


def kernel(x, doc_features, p, W_ih, W_hh, b_ih, b_hh, W0, W_doc, b_doc,
           bn_gamma, bn_beta, W_fuse, b_fuse, W_task, b_task, W_time, b_time,
           edge_index, batch, bn_mean, bn_var):
    n, d = x.shape
    nb = doc_features.shape[0]
    e = edge_index.shape[1]
    f32 = jnp.float32

    np_ = _rup(n + 1, _NS * _CH)
    ep = _rup(e, _NW * _CH)
    cwh = ep // (_NW * _CH)
    rf = _rup(n, _CH) // _CH
    cw0 = cwh
    cwmax = cwh

    src = jnp.concatenate(
        [edge_index[0], jnp.full((ep - e,), 0, edge_index.dtype)]
    ).reshape(_NW, cwh, _CH)
    dst = jnp.concatenate(
        [edge_index[1], jnp.full((ep - e,), n, edge_index.dtype)]
    ).reshape(_NW, cwh, _CH)
    dst_h = dst
    xpad = jnp.pad(x, ((0, rf * _CH - n), (0, 0)))
    bih2 = b_ih.reshape(1, -1)
    bhh2 = b_hh.reshape(1, -1)
    batchp = jnp.concatenate(
        [batch, jnp.full((np_ - n,), nb, batch.dtype)]).reshape(1, np_)

    mesh = plsc.VectorSubcoreMesh(core_axis_name="c", subcore_axis_name="s",
                                  num_cores=_NC, num_subcores=_NS)

    hist = pl.kernel(
        functools.partial(_sc_hist_body, np_, cwh),
        out_type=jax.ShapeDtypeStruct((_NC, np_, _DEGW), f32),
        mesh=mesh,
        scratch_types=[
            pltpu.VMEM((cwh, _CH), jnp.int32),
            pltpu.VMEM((_CH, _DEGW), f32),
            pltpu.VMEM_SHARED((np_, _DEGW), f32),
        ],
        name="sc_deg_hist",
    )
    deg2 = hist(dst_h)

    score = pl.pallas_call(
        _tc_score_body,
        out_shape=jax.ShapeDtypeStruct((rf * _CH, 1), f32),
        name="tc_score",
    )
    pn = (jnp.linalg.norm(p) + 1e-16).reshape(1, 1)
    sc_fold = score(xpad, p.reshape(d, 1), pn).reshape(rf, _CH)

    dense = pl.pallas_call(
        functools.partial(_tc_dense_body, n, rf),
        out_shape=jax.ShapeDtypeStruct((n, d), f32),
        scratch_shapes=[
            pltpu.VMEM((rf, _CH), f32),
            pltpu.VMEM((d, d), f32),
        ],
        name="tc_dense",
    )
    xw = dense(x, sc_fold, W_ih, W_hh, bih2, bhh2, W0)

    scale = pl.pallas_call(
        _tc_scale_body,
        out_shape=(jax.ShapeDtypeStruct((n, d), f32),
                   jax.ShapeDtypeStruct((n, 1), f32)),
        name="tc_scale",
    )
    xs, dis = scale(xw, deg2[0, :n, 0:1], deg2[1, :n, 0:1])

    xs_p = jnp.pad(xs, ((0, np_ - n), (0, 0)))
    dis_p = jnp.pad(dis, ((0, np_ - n), (0, 0)))

    scat = pl.kernel(
        functools.partial(_sc_scatter_body, np_, cw0, d),
        out_type=jax.ShapeDtypeStruct((_NC, np_, d), f32),
        mesh=mesh,
        scratch_types=[
            pltpu.VMEM((cwmax, _CH), jnp.int32),
            pltpu.VMEM((cwmax, _CH), jnp.int32),
            pltpu.VMEM((_CH, d), f32),
            pltpu.VMEM_SHARED((np_, d), f32),
            pltpu.SemaphoreType.DMA,
        ],
        name="sc_edge_scatter",
    )
    s2 = scat(xs_p, src, dst)

    n_task = W_task.shape[0]
    n_time = W_time.shape[0]
    w_head = jnp.concatenate([W_task, W_time], axis=0)
    b_head = jnp.concatenate([b_task, b_time]).reshape(1, -1)
    final = pl.pallas_call(
        functools.partial(_tc_final_body, nb),
        out_shape=jax.ShapeDtypeStruct((nb, n_task + n_time), f32),
        name="tc_final",
    )
    heads = final(s2, xs_p, dis_p, batchp, doc_features, W_doc,
                  b_doc.reshape(1, -1), bn_gamma.reshape(1, -1),
                  bn_beta.reshape(1, -1), bn_mean.reshape(1, -1),
                  bn_var.reshape(1, -1), W_fuse, b_fuse.reshape(1, -1),
                  w_head, b_head)
    return (heads[:, :n_task], heads[:, n_task:])
